# baseline (speedup 1.0000x reference)
import functools

import jax
import jax.numpy as jnp
from jax import lax
from jax.experimental import pallas as pl
from jax.experimental.pallas import tpu as pltpu
from jax.experimental.pallas import tpu_sc as plsc

N_NODES = 10000
N_PAD = 10240
E_EDGES = 320000
D_FEAT = 128
D_HALF = D_FEAT // 2
D_EDGE = 16
D_OUT = 128

NC = 2
NS = 16
CHUNK = 80
EPT = E_EDGES // NS
CPT = EPT // CHUNK
NSLOT = 2
ITERS = CPT // NSLOT
EF_ITERS = ITERS // 2
ROWS_PER_TILE = N_PAD // NS

_MESH = plsc.VectorSubcoreMesh(core_axis_name="c", subcore_axis_name="s")


@functools.partial(
    pl.kernel,
    out_type=(
        jax.ShapeDtypeStruct((NC * N_PAD, D_HALF), jnp.float32),
        jax.ShapeDtypeStruct((NC * N_PAD, D_EDGE), jnp.float32),
        jax.ShapeDtypeStruct((NC * N_NODES, D_HALF), jnp.float32),
    ),
    mesh=_MESH,
    compiler_params=pltpu.CompilerParams(use_tc_tiling_on_sc=False),
    scratch_types=[
        pltpu.VMEM((EPT,), jnp.int32),
        pltpu.VMEM((EPT,), jnp.int32),
        [pltpu.VMEM((CHUNK,), jnp.int32)] * NSLOT,
        [pltpu.VMEM((CHUNK, D_HALF), jnp.float32)] * NSLOT,
        [pltpu.VMEM((CHUNK * D_EDGE,), jnp.float32)] * NSLOT,
        [pltpu.VMEM((CHUNK, D_EDGE), jnp.float32)] * NSLOT,
        pltpu.VMEM((ROWS_PER_TILE, D_EDGE), jnp.float32),
        pltpu.VMEM_SHARED((N_PAD, D_HALF), jnp.float32),
        pltpu.VMEM_SHARED((N_PAD, D_EDGE), jnp.float32),
        [pltpu.SemaphoreType.DMA] * NSLOT,
        [pltpu.SemaphoreType.DMA] * NSLOT,
        [pltpu.SemaphoreType.DMA] * NSLOT,
        [pltpu.SemaphoreType.DMA] * NSLOT,
    ],
)
def _sc_aggregate(srclo_hbm, srchi_hbm, dst_hbm, x_hbm, ef_hbm,
                  zx_hbm, ze_hbm, outx_hbm, oute_hbm, xc_hbm,
                  sidx_v, didx_v, dstv, rows, efl, efv, eb_v,
                  aggx_s, agge_s, sem_g, sem_s, sem_el, sem_es):
    c = lax.axis_index("c")
    s = lax.axis_index("s")

    @pl.when(c == 0)
    def _():
        pltpu.sync_copy(srclo_hbm.at[pl.ds(s * EPT, EPT)], sidx_v)

    @pl.when(c == 1)
    def _():
        pltpu.sync_copy(srchi_hbm.at[pl.ds(s * EPT, EPT)], sidx_v)

    pltpu.sync_copy(dst_hbm.at[pl.ds(s * EPT, EPT)], didx_v)

    row0 = s * (N_NODES // NS)
    for k in range(8):
        nr = 80 if k < 7 else 65
        rb = row0 + k * 80

        @pl.when(c == 0)
        def _(nr=nr, rb=rb):
            pltpu.sync_copy(x_hbm.at[pl.ds(rb, nr), pl.ds(0, D_HALF)],
                            rows[0].at[pl.ds(0, nr)])

        @pl.when(c == 1)
        def _(nr=nr, rb=rb):
            pltpu.sync_copy(x_hbm.at[pl.ds(rb, nr), pl.ds(D_HALF, D_HALF)],
                            rows[0].at[pl.ds(0, nr)])

        pltpu.sync_copy(rows[0].at[pl.ds(0, nr)],
                        xc_hbm.at[pl.ds(c * N_NODES + rb, nr)])

    r0 = s * ROWS_PER_TILE
    pltpu.sync_copy(zx_hbm.at[pl.ds(0, CHUNK)], rows[0])
    pltpu.sync_copy(ze_hbm.at[pl.ds(0, ROWS_PER_TILE)], eb_v)
    for k in range(ROWS_PER_TILE // CHUNK):
        pltpu.sync_copy(rows[0], aggx_s.at[pl.ds(r0 + k * CHUNK, CHUNK)])
    pltpu.sync_copy(eb_v, agge_s.at[pl.ds(r0, ROWS_PER_TILE)])
    plsc.subcore_barrier()

    ef_base = s * EPT

    def gather_wait(j, g):
        pltpu.make_async_copy(
            xc_hbm.at[sidx_v.at[pl.ds(g * CHUNK, CHUNK)]], rows[j],
            sem_g[j]).wait()

    def gather_issue(j, g):
        for t in range(CHUNK // 16):
            dstv[j][pl.ds(16 * t, 16)] = didx_v[pl.ds(g * CHUNK + 16 * t, 16)]
        pltpu.async_copy(
            xc_hbm.at[sidx_v.at[pl.ds(g * CHUNK, CHUNK)]], rows[j], sem_g[j])

    def ef_issue(j, g):
        pltpu.async_copy(
            ef_hbm.at[pl.ds((ef_base + g * CHUNK) * D_EDGE, CHUNK * D_EDGE)],
            efl[j], sem_el[j])

    for j in range(NSLOT):
        gather_issue(j, j)

    @pl.when(c == 0)
    def _():
        for j in range(NSLOT):
            ef_issue(j, j)

    def body(i, carry):
        own_ef = (i < EF_ITERS) == (c == 0)
        own_ef_next = ((i + 1) < EF_ITERS) == (c == 0)

        for j in range(NSLOT):
            g = i * NSLOT + j
            gather_wait(j, g)
            pltpu.async_copy(rows[j], aggx_s.at[dstv[j]], sem_s[j], add=True)

            @pl.when(own_ef)
            def _(j=j):
                pltpu.make_async_copy(
                    ef_hbm.at[pl.ds(0, CHUNK * D_EDGE)], efl[j],
                    sem_el[j]).wait()
                for e in range(CHUNK):
                    efv[j][e, :] = efl[j][pl.ds(e * D_EDGE, D_EDGE)]
                pltpu.async_copy(efv[j], agge_s.at[dstv[j]], sem_es[j],
                                 add=True)

        @pl.when(i + 1 < ITERS)
        def _():
            for j in range(NSLOT):
                gn = (i + 1) * NSLOT + j
                pltpu.make_async_copy(rows[j], aggx_s.at[dstv[j]],
                                      sem_s[j]).wait()

                @pl.when(own_ef)
                def _(j=j):
                    pltpu.make_async_copy(efv[j], agge_s.at[dstv[j]],
                                          sem_es[j]).wait()

                gather_issue(j, gn)

                @pl.when(own_ef_next)
                def _(j=j, gn=gn):
                    ef_issue(j, gn)

        return carry

    lax.fori_loop(0, ITERS, body, 0)

    for j in range(NSLOT):
        pltpu.make_async_copy(rows[j], aggx_s.at[dstv[j]], sem_s[j]).wait()

        @pl.when(c == 1)
        def _(j=j):
            pltpu.make_async_copy(efv[j], agge_s.at[dstv[j]],
                                  sem_es[j]).wait()

    plsc.subcore_barrier()

    out_base = c * N_PAD + r0
    for k in range(ROWS_PER_TILE // CHUNK):
        pltpu.sync_copy(aggx_s.at[pl.ds(r0 + k * CHUNK, CHUNK)], rows[0])
        pltpu.sync_copy(rows[0], outx_hbm.at[pl.ds(out_base + k * CHUNK,
                                                   CHUNK)])
    pltpu.sync_copy(agge_s.at[pl.ds(r0, ROWS_PER_TILE)], eb_v)
    pltpu.sync_copy(eb_v, oute_hbm.at[pl.ds(out_base, ROWS_PER_TILE)])


BLK = 1280


def _mm_body(axl_ref, axh_ref, ae0_ref, ae1_ref, wxl_ref, wxh_ref, we_ref,
             b_ref, o_ref):
    ae = ae0_ref[...] + ae1_ref[...]
    acc = lax.dot_general(axl_ref[...], wxl_ref[...], (((1,), (0,)), ((), ())),
                          preferred_element_type=jnp.float32,
                          precision=lax.Precision.HIGHEST)
    acc = acc + lax.dot_general(axh_ref[...], wxh_ref[...],
                                (((1,), (0,)), ((), ())),
                                preferred_element_type=jnp.float32,
                                precision=lax.Precision.HIGHEST)
    acc = acc + lax.dot_general(ae, we_ref[...], (((1,), (0,)), ((), ())),
                                preferred_element_type=jnp.float32,
                                precision=lax.Precision.HIGHEST)
    o_ref[...] = acc + b_ref[...]


def kernel(node_feature, edge_index, edge_feature, W, b):
    edge_index = edge_index.astype(jnp.int32)
    srclo = jnp.ravel(edge_index[0])
    srchi = srclo + N_NODES
    dst = jnp.ravel(edge_index[1])
    zx = jnp.zeros((CHUNK, D_HALF), jnp.float32)
    ze = jnp.zeros((ROWS_PER_TILE, D_EDGE), jnp.float32)
    outx, oute, _ = _sc_aggregate(srclo, srchi, dst, node_feature,
                                  edge_feature.reshape(-1), zx, ze)
    wxl = W[:, :D_HALF].T
    wxh = W[:, D_HALF:D_FEAT].T
    we = W[:, D_FEAT:].T
    out = pl.pallas_call(
        _mm_body,
        grid=(N_PAD // BLK,),
        in_specs=[
            pl.BlockSpec((BLK, D_HALF), lambda i: (i, 0)),
            pl.BlockSpec((BLK, D_HALF), lambda i: (N_PAD // BLK + i, 0)),
            pl.BlockSpec((BLK, D_EDGE), lambda i: (i, 0)),
            pl.BlockSpec((BLK, D_EDGE), lambda i: (N_PAD // BLK + i, 0)),
            pl.BlockSpec((D_HALF, D_OUT), lambda i: (0, 0)),
            pl.BlockSpec((D_HALF, D_OUT), lambda i: (0, 0)),
            pl.BlockSpec((D_EDGE, D_OUT), lambda i: (0, 0)),
            pl.BlockSpec((1, D_OUT), lambda i: (0, 0)),
        ],
        out_specs=pl.BlockSpec((BLK, D_OUT), lambda i: (i, 0)),
        out_shape=jax.ShapeDtypeStruct((N_PAD, D_OUT), jnp.float32),
    )(outx, outx, oute, oute, wxl, wxh, we, b.reshape(1, D_OUT))
    return out[:N_NODES]


# speedup vs baseline: 1.4981x; 1.4981x over previous
import functools

import jax
import jax.numpy as jnp
from jax import lax
from jax.experimental import pallas as pl
from jax.experimental.pallas import tpu as pltpu
from jax.experimental.pallas import tpu_sc as plsc

N_NODES = 10000
N_PAD = 10240
E_EDGES = 320000
D_FEAT = 128
D_HALF = D_FEAT // 2
D_EDGE = 16
D_OUT = 128

NC = 2
NS = 16
CHUNK = 80

EPT = E_EDGES // NS
CPT = EPT // CHUNK
NSLOT = 5
ITERS = CPT // NSLOT

EPT_B = E_EDGES // (NC * NS)
CPT_B = EPT_B // CHUNK
NSLOT_B = 5
ITERS_B = CPT_B // NSLOT_B

ROWS_PER_TILE = N_PAD // NS
NROWS_PT = N_NODES // NS

_MESH = plsc.VectorSubcoreMesh(core_axis_name="c", subcore_axis_name="s")


@functools.partial(
    pl.kernel,
    out_type=(
        jax.ShapeDtypeStruct((NC * N_PAD, D_HALF), jnp.float32),
        jax.ShapeDtypeStruct((NC * N_NODES, D_HALF), jnp.float32),
    ),
    mesh=_MESH,
    compiler_params=pltpu.CompilerParams(use_tc_tiling_on_sc=False),
    scratch_types=[
        pltpu.VMEM((EPT,), jnp.int32),
        pltpu.VMEM((EPT,), jnp.int32),
        [pltpu.VMEM((CHUNK,), jnp.int32)] * NSLOT,
        [pltpu.VMEM((CHUNK, D_HALF), jnp.float32)] * NSLOT,
        pltpu.VMEM_SHARED((N_PAD, D_HALF), jnp.float32),
        [pltpu.SemaphoreType.DMA] * NSLOT,
        [pltpu.SemaphoreType.DMA] * NSLOT,
    ],
)
def _sc_agg_x(srclo_hbm, srchi_hbm, dst_hbm, x_hbm, zx_hbm,
              outx_hbm, xc_hbm,
              sidx_v, didx_v, dstv, rows, aggx_s, sem_g, sem_s):
    c = lax.axis_index("c")
    s = lax.axis_index("s")

    @pl.when(c == 0)
    def _():
        pltpu.sync_copy(srclo_hbm.at[pl.ds(s * EPT, EPT)], sidx_v)

    @pl.when(c == 1)
    def _():
        pltpu.sync_copy(srchi_hbm.at[pl.ds(s * EPT, EPT)], sidx_v)

    pltpu.sync_copy(dst_hbm.at[pl.ds(s * EPT, EPT)], didx_v)

    row0 = s * NROWS_PT
    for k in range(8):
        nr = 80 if k < 7 else 65
        rb = row0 + k * 80

        @pl.when(c == 0)
        def _(nr=nr, rb=rb):
            pltpu.sync_copy(x_hbm.at[pl.ds(rb, nr), pl.ds(0, D_HALF)],
                            rows[0].at[pl.ds(0, nr)])

        @pl.when(c == 1)
        def _(nr=nr, rb=rb):
            pltpu.sync_copy(x_hbm.at[pl.ds(rb, nr), pl.ds(D_HALF, D_HALF)],
                            rows[0].at[pl.ds(0, nr)])

        pltpu.sync_copy(rows[0].at[pl.ds(0, nr)],
                        xc_hbm.at[pl.ds(c * N_NODES + rb, nr)])

    r0 = s * ROWS_PER_TILE
    pltpu.sync_copy(zx_hbm.at[pl.ds(0, CHUNK)], rows[0])
    for k in range(ROWS_PER_TILE // CHUNK):
        pltpu.sync_copy(rows[0], aggx_s.at[pl.ds(r0 + k * CHUNK, CHUNK)])
    plsc.subcore_barrier()

    def gather_wait(j, g):
        pltpu.make_async_copy(
            xc_hbm.at[sidx_v.at[pl.ds(g * CHUNK, CHUNK)]], rows[j],
            sem_g[j]).wait()

    def gather_issue(j, g):
        for t in range(CHUNK // 16):
            dstv[j][pl.ds(16 * t, 16)] = didx_v[pl.ds(g * CHUNK + 16 * t, 16)]
        pltpu.async_copy(
            xc_hbm.at[sidx_v.at[pl.ds(g * CHUNK, CHUNK)]], rows[j], sem_g[j])

    for j in range(NSLOT):
        gather_issue(j, j)

    def body(i, carry):
        for j in range(NSLOT):
            g = i * NSLOT + j
            gather_wait(j, g)
            pltpu.async_copy(rows[j], aggx_s.at[dstv[j]], sem_s[j], add=True)

        @pl.when(i + 1 < ITERS)
        def _():
            for j in range(NSLOT):
                gn = (i + 1) * NSLOT + j
                pltpu.make_async_copy(rows[j], aggx_s.at[dstv[j]],
                                      sem_s[j]).wait()
                gather_issue(j, gn)

        return carry

    lax.fori_loop(0, ITERS, body, 0)

    for j in range(NSLOT):
        pltpu.make_async_copy(rows[j], aggx_s.at[dstv[j]], sem_s[j]).wait()

    plsc.subcore_barrier()

    out_base = c * N_PAD + r0
    for k in range(ROWS_PER_TILE // CHUNK):
        pltpu.sync_copy(aggx_s.at[pl.ds(r0 + k * CHUNK, CHUNK)], rows[0])
        pltpu.sync_copy(rows[0], outx_hbm.at[pl.ds(out_base + k * CHUNK,
                                                   CHUNK)])


@functools.partial(
    pl.kernel,
    out_type=jax.ShapeDtypeStruct((NC * N_PAD, D_EDGE), jnp.float32),
    mesh=_MESH,
    compiler_params=pltpu.CompilerParams(use_tc_tiling_on_sc=False),
    scratch_types=[
        pltpu.VMEM((EPT_B,), jnp.int32),
        [pltpu.VMEM((CHUNK,), jnp.int32)] * NSLOT_B,
        [pltpu.VMEM((CHUNK * D_EDGE,), jnp.float32)] * NSLOT_B,
        [pltpu.VMEM((CHUNK, D_EDGE), jnp.float32)] * NSLOT_B,
        pltpu.VMEM((ROWS_PER_TILE, D_EDGE), jnp.float32),
        pltpu.VMEM_SHARED((N_PAD, D_EDGE), jnp.float32),
        [pltpu.SemaphoreType.DMA] * NSLOT_B,
        [pltpu.SemaphoreType.DMA] * NSLOT_B,
    ],
)
def _sc_agg_ef(dst_hbm, ef_hbm, ze_hbm, oute_hbm,
               didx_v, dstv, efl, efv, eb_v, agge_s, sem_el, sem_es):
    c = lax.axis_index("c")
    s = lax.axis_index("s")
    w = c * NS + s

    pltpu.sync_copy(dst_hbm.at[pl.ds(w * EPT_B, EPT_B)], didx_v)

    r0 = s * ROWS_PER_TILE
    pltpu.sync_copy(ze_hbm.at[pl.ds(0, ROWS_PER_TILE)], eb_v)
    pltpu.sync_copy(eb_v, agge_s.at[pl.ds(r0, ROWS_PER_TILE)])
    plsc.subcore_barrier()

    def ef_issue(j, g):
        for t in range(CHUNK // 16):
            dstv[j][pl.ds(16 * t, 16)] = didx_v[pl.ds(g * CHUNK + 16 * t, 16)]
        pltpu.async_copy(
            ef_hbm.at[pl.ds((w * EPT_B + g * CHUNK) * D_EDGE,
                            CHUNK * D_EDGE)],
            efl[j], sem_el[j])

    for j in range(NSLOT_B):
        ef_issue(j, j)

    def body(i, carry):
        for j in range(NSLOT_B):
            pltpu.make_async_copy(
                ef_hbm.at[pl.ds(0, CHUNK * D_EDGE)], efl[j],
                sem_el[j]).wait()
            for e in range(CHUNK):
                efv[j][e, :] = efl[j][pl.ds(e * D_EDGE, D_EDGE)]
            pltpu.async_copy(efv[j], agge_s.at[dstv[j]], sem_es[j], add=True)

        @pl.when(i + 1 < ITERS_B)
        def _():
            for j in range(NSLOT_B):
                gn = (i + 1) * NSLOT_B + j
                pltpu.make_async_copy(efv[j], agge_s.at[dstv[j]],
                                      sem_es[j]).wait()
                ef_issue(j, gn)

        return carry

    lax.fori_loop(0, ITERS_B, body, 0)

    for j in range(NSLOT_B):
        pltpu.make_async_copy(efv[j], agge_s.at[dstv[j]], sem_es[j]).wait()

    plsc.subcore_barrier()

    out_base = c * N_PAD + r0
    pltpu.sync_copy(agge_s.at[pl.ds(r0, ROWS_PER_TILE)], eb_v)
    pltpu.sync_copy(eb_v, oute_hbm.at[pl.ds(out_base, ROWS_PER_TILE)])


BLK = 1280


def _mm_body(axl_ref, axh_ref, ae0_ref, ae1_ref, wxl_ref, wxh_ref, we_ref,
             b_ref, o_ref):
    ae = ae0_ref[...] + ae1_ref[...]
    acc = lax.dot_general(axl_ref[...], wxl_ref[...], (((1,), (0,)), ((), ())),
                          preferred_element_type=jnp.float32,
                          precision=lax.Precision.HIGHEST)
    acc = acc + lax.dot_general(axh_ref[...], wxh_ref[...],
                                (((1,), (0,)), ((), ())),
                                preferred_element_type=jnp.float32,
                                precision=lax.Precision.HIGHEST)
    acc = acc + lax.dot_general(ae, we_ref[...], (((1,), (0,)), ((), ())),
                                preferred_element_type=jnp.float32,
                                precision=lax.Precision.HIGHEST)
    o_ref[...] = acc + b_ref[...]


def kernel(node_feature, edge_index, edge_feature, W, b):
    edge_index = edge_index.astype(jnp.int32)
    srclo = jnp.ravel(edge_index[0])
    srchi = srclo + N_NODES
    dst = jnp.ravel(edge_index[1])
    zx = jnp.zeros((CHUNK, D_HALF), jnp.float32)
    ze = jnp.zeros((ROWS_PER_TILE, D_EDGE), jnp.float32)
    outx, _ = _sc_agg_x(srclo, srchi, dst, node_feature, zx)
    oute = _sc_agg_ef(dst, edge_feature.reshape(-1), ze)
    wxl = W[:, :D_HALF].T
    wxh = W[:, D_HALF:D_FEAT].T
    we = W[:, D_FEAT:].T
    out = pl.pallas_call(
        _mm_body,
        grid=(N_PAD // BLK,),
        in_specs=[
            pl.BlockSpec((BLK, D_HALF), lambda i: (i, 0)),
            pl.BlockSpec((BLK, D_HALF), lambda i: (N_PAD // BLK + i, 0)),
            pl.BlockSpec((BLK, D_EDGE), lambda i: (i, 0)),
            pl.BlockSpec((BLK, D_EDGE), lambda i: (N_PAD // BLK + i, 0)),
            pl.BlockSpec((D_HALF, D_OUT), lambda i: (0, 0)),
            pl.BlockSpec((D_HALF, D_OUT), lambda i: (0, 0)),
            pl.BlockSpec((D_EDGE, D_OUT), lambda i: (0, 0)),
            pl.BlockSpec((1, D_OUT), lambda i: (0, 0)),
        ],
        out_specs=pl.BlockSpec((BLK, D_OUT), lambda i: (i, 0)),
        out_shape=jax.ShapeDtypeStruct((N_PAD, D_OUT), jnp.float32),
    )(outx, outx, oute, oute, wxl, wxh, we, b.reshape(1, D_OUT))
    return out[:N_NODES]
